# baseline (speedup 1.0000x reference)
import functools

import jax
import jax.numpy as jnp
from jax import lax
from jax.experimental import pallas as pl
from jax.experimental.pallas import tpu as pltpu
from jax.experimental.pallas import tpu_sc as plsc

_NW = 32
_B = 16384 * 32
_D = 128
_V = 32
_G = 128
_PER_W = _B // _NW
_NGRP = _PER_W // _G
_NB = 3
_NT = _NGRP // (2 * _NB)
_TAIL = _NGRP - 2 * _NB * _NT

_mesh = plsc.VectorSubcoreMesh(core_axis_name="c", subcore_axis_name="s")


@functools.partial(
    pl.kernel,
    mesh=_mesh,
    out_type=jax.ShapeDtypeStruct((_B, _D), jnp.float32),
    compiler_params=pltpu.CompilerParams(needs_layout_passes=False),
    scratch_types=[
        pltpu.VMEM((_NGRP, _G), jnp.int32),
        pltpu.VMEM((_NB * _G, _D), jnp.float32),
        pltpu.VMEM((_NB * _G, _D), jnp.float32),
        pltpu.VMEM_SHARED((_V, _D), jnp.float32),
        pltpu.SemaphoreType.DMA,
        pltpu.SemaphoreType.DMA,
        pltpu.SemaphoreType.DMA,
        pltpu.SemaphoreType.DMA,
    ],
)
def _gather_all(idx_hbm, table_hbm, out_hbm, idx_v, ba, bb, tab_sh,
                ga_sem, gb_sem, wa_sem, wb_sem):
    sid = lax.axis_index("s")
    wid = sid * 2 + lax.axis_index("c")
    base = wid * _PER_W

    @pl.when(sid == 0)
    def _():
        pltpu.sync_copy(table_hbm, tab_sh)

    pltpu.sync_copy(idx_hbm.at[wid], idx_v)
    plsc.subcore_barrier()

    bufs = {"a": ba, "b": bb}
    gsems = {"a": ga_sem, "b": gb_sem}
    wsems = {"a": wa_sem, "b": wb_sem}

    def g_start(k, ga):
        for i in range(_NB):
            g = jnp.minimum(ga + i, _NGRP - 1)
            pltpu.async_copy(tab_sh.at[idx_v.at[g]],
                             bufs[k].at[pl.ds(i * _G, _G)], gsems[k])

    def g_wait(k):
        for i in range(_NB):
            pltpu.make_async_copy(tab_sh.at[idx_v.at[0]],
                                  bufs[k].at[pl.ds(i * _G, _G)],
                                  gsems[k]).wait()

    def w_start(k, ga, n=_NB):
        pltpu.async_copy(bufs[k].at[pl.ds(0, n * _G)],
                         out_hbm.at[pl.ds(base + ga * _G, n * _G)], wsems[k])

    def w_wait(k, n=_NB):
        pltpu.make_async_copy(bufs[k].at[pl.ds(0, n * _G)],
                              out_hbm.at[pl.ds(base, n * _G)],
                              wsems[k]).wait()

    g_start("a", 0)
    g_wait("a")
    w_start("a", 0)
    g_start("b", _NB)
    g_wait("b")
    w_start("b", _NB)
    w_wait("a")
    g_start("a", 2 * _NB)

    def body(t, carry):
        ga = 2 * _NB * t
        g_wait("a")
        w_start("a", ga)
        w_wait("b")
        g_start("b", ga + _NB)
        g_wait("b")
        w_start("b", ga + _NB)
        w_wait("a")
        g_start("a", ga + 2 * _NB)
        return carry

    lax.fori_loop(1, _NT, body, 0)

    g_wait("a")
    w_start("a", 2 * _NB * _NT, n=_TAIL)
    w_wait("b")
    w_wait("a", n=_TAIL)


def kernel(x, weight):
    xr = x.reshape(_NW, _NGRP, _G)
    out = _gather_all(xr, weight)
    return out.reshape(16384, 32, _D)
